# baseline (speedup 1.0000x reference)
import functools

import jax
import jax.numpy as jnp
from jax import lax
from jax.experimental import pallas as pl
from jax.experimental.pallas import tpu as pltpu
from jax.experimental.pallas import tpu_sc as plsc

E = 16
H = 1024
F = 1024
D = 2048
R = 4
TOP_K = 2
T = 2048
A = T * TOP_K
TM = 128
NB = E + A // TM
NS = NB * TM
CH = 256
NCH = A // CH
SCALING = 1.0 / R
LIMIT = 7.0
GLU_ALPHA = 1.702

NBA = 56
NW = 32
TPW = T // NW


def _k1_body(x_ref, pos_ref, blk_ref, cnt_ref, carry_ref, st_ref):
    p = pl.program_id(0)
    r = pl.program_id(1)
    x = x_ref[0]
    ei = lax.broadcasted_iota(jnp.int32, (E, CH), 0)
    a16 = (ei == x).astype(jnp.float32)

    @pl.when(p == 0)
    def _():
        ccnt = jnp.sum(a16, axis=1, keepdims=True)
        cnt_ref[...] = jnp.where(r == 0, ccnt, cnt_ref[...] + ccnt)

    @pl.when((p == 1) & (r == 0))
    def _():
        cnt = cnt_ref[...]
        nb = jnp.floor((cnt + (TM - 1.0)) / TM)
        li = lax.broadcasted_iota(jnp.int32, (E, E), 0)
        lj = lax.broadcasted_iota(jnp.int32, (E, E), 1)
        ltri = (lj < li).astype(jnp.float32)
        sb = jnp.dot(ltri, nb, preferred_element_type=jnp.float32)
        st_ref[...] = TM * sb
        carry_ref[...] = jnp.zeros_like(carry_ref)
        bi = lax.broadcasted_iota(jnp.int32, (E, NBA), 1).astype(jnp.float32)
        owns = (sb <= bi).astype(jnp.float32)
        bexp = jnp.sum(owns, axis=0, keepdims=True) - 1.0
        used = jnp.sum(nb)
        ci = lax.broadcasted_iota(jnp.int32, (1, NBA), 1)
        blk_ref[...] = jnp.where(ci == NB, used, bexp).astype(jnp.int32)

    @pl.when(p == 1)
    def _():
        ji = lax.broadcasted_iota(jnp.int32, (CH, CH), 0)
        jc = lax.broadcasted_iota(jnp.int32, (CH, CH), 1)
        utri = (ji < jc).astype(jnp.float32)
        ranks = jnp.dot(a16, utri, preferred_element_type=jnp.float32)
        base = carry_ref[...] + st_ref[...]
        pos = jnp.sum(a16 * (ranks + base), axis=0, keepdims=True)
        pos_ref[...] = pos.astype(jnp.int32).reshape(1, 1, CH)
        carry_ref[...] = carry_ref[...] + jnp.sum(a16, axis=1, keepdims=True)


def _routing_tc(x32, interpret=False):
    return pl.pallas_call(
        _k1_body,
        grid=(2, NCH),
        in_specs=[pl.BlockSpec((1, 1, CH), lambda p, r: (r, 0, 0))],
        out_specs=[
            pl.BlockSpec((1, 1, CH), lambda p, r: (r, 0, 0)),
            pl.BlockSpec((1, NBA), lambda p, r: (0, 0)),
        ],
        out_shape=[
            jax.ShapeDtypeStruct((NCH, 1, CH), jnp.int32),
            jax.ShapeDtypeStruct((1, NBA), jnp.int32),
        ],
        scratch_shapes=[
            pltpu.VMEM((E, 1), jnp.float32),
            pltpu.VMEM((E, 1), jnp.float32),
            pltpu.VMEM((E, 1), jnp.float32),
        ],
        interpret=interpret,
    )(x32)


def _k3_body(blk_ref, x_ref, w1_ref, w2_ref, a1_ref, b1_ref,
             a2_ref, b2_ref, b1b_ref, bd_ref, s_ref, y_ref):
    b = pl.program_id(0)

    @pl.when(b < blk_ref[NB])
    def _():
        x = x_ref[...]
        xb = x.astype(jnp.bfloat16)
        w1 = w1_ref[0].astype(jnp.bfloat16)
        mid = jnp.dot(xb, a1_ref[0].astype(jnp.bfloat16),
                      preferred_element_type=jnp.float32)
        midb = (mid * SCALING).astype(jnp.bfloat16)
        gu = (jnp.dot(xb, w1, preferred_element_type=jnp.float32)
              + jnp.dot(midb, b1_ref[0].astype(jnp.bfloat16),
                        preferred_element_type=jnp.float32)
              + b1b_ref[0])
        ur = jnp.concatenate([gu[:, 1:], gu[:, :1]], axis=1)
        g = jnp.minimum(gu, LIMIT)
        u = jnp.clip(ur, -LIMIT, LIMIT)
        glu = g * (1.0 / (1.0 + jnp.exp(-GLU_ALPHA * g)))
        gatedi = ((u + 1.0) * glu).astype(jnp.bfloat16)
        gatedb = jnp.dot(gatedi, s_ref[...],
                         preferred_element_type=jnp.float32).astype(jnp.bfloat16)
        mid2 = jnp.dot(gatedb, a2_ref[0].astype(jnp.bfloat16),
                       preferred_element_type=jnp.float32)
        mid2b = (mid2 * SCALING).astype(jnp.bfloat16)
        y = (jnp.dot(gatedb, w2_ref[0].astype(jnp.bfloat16),
                     preferred_element_type=jnp.float32)
             + jnp.dot(mid2b, b2_ref[0].astype(jnp.bfloat16),
                       preferred_element_type=jnp.float32)
             + bd_ref[0])
        y_ref[...] = y


def _mlp_tc(blk, xg, w1, w2, a1, b1, a2, b2, b1b, bd, sel, interpret=False):
    eix = lambda b, blk: (blk[b], 0, 0)
    grid_spec = pltpu.PrefetchScalarGridSpec(
        num_scalar_prefetch=1,
        grid=(NB,),
        in_specs=[
            pl.BlockSpec((TM, H), lambda b, blk: (b, 0)),
            pl.BlockSpec((1, H, D), eix),
            pl.BlockSpec((1, F, H), eix),
            pl.BlockSpec((1, H, R), eix),
            pl.BlockSpec((1, R, D), eix),
            pl.BlockSpec((1, F, R), eix),
            pl.BlockSpec((1, R, H), eix),
            pl.BlockSpec((1, 1, D), eix),
            pl.BlockSpec((1, 1, H), eix),
            pl.BlockSpec((D, F), lambda b, blk: (0, 0)),
        ],
        out_specs=pl.BlockSpec((TM, H), lambda b, blk: (b, 0)),
    )
    return pl.pallas_call(
        _k3_body,
        grid_spec=grid_spec,
        out_shape=jax.ShapeDtypeStruct((NS, H), jnp.float32),
        interpret=interpret,
    )(blk, xg, w1, w2, a1, b1, a2, b2, b1b, bd, sel)


def _dispatch_sc(hs, pos_flat):
    mesh = plsc.VectorSubcoreMesh(core_axis_name="c", subcore_axis_name="s")

    @functools.partial(
        pl.kernel,
        mesh=mesh,
        out_type=jax.ShapeDtypeStruct((NS, H), jnp.float32),
        scratch_types=[
            pltpu.VMEM((TPW, H), jnp.float32),
            pltpu.VMEM((TPW,), jnp.int32),
            pltpu.VMEM((TPW,), jnp.int32),
            pltpu.SemaphoreType.DMA,
        ],
    )
    def k2(hs_hbm, pos_hbm, xg_hbm, rows_v, idx0_v, idx1_v, sem):
        wid = lax.axis_index("s") * 2 + lax.axis_index("c")
        base = wid * TPW
        pltpu.sync_copy(hs_hbm.at[pl.ds(base, TPW)], rows_v)
        pltpu.sync_copy(pos_hbm.at[pl.ds(base, TPW)], idx0_v)
        pltpu.sync_copy(pos_hbm.at[pl.ds(T + base, TPW)], idx1_v)
        pltpu.async_copy(rows_v, xg_hbm.at[idx0_v], sem).wait()
        pltpu.async_copy(rows_v, xg_hbm.at[idx1_v], sem).wait()

    return k2(hs, pos_flat)


CHT = 32


def _combine_sc(yg, pos_flat, rw_flat):
    mesh = plsc.VectorSubcoreMesh(core_axis_name="c", subcore_axis_name="s")

    @functools.partial(
        pl.kernel,
        mesh=mesh,
        out_type=jax.ShapeDtypeStruct((T, H), jnp.float32),
        scratch_types=[
            pltpu.VMEM((CHT, H), jnp.float32),
            pltpu.VMEM((CHT, H), jnp.float32),
            pltpu.VMEM((CHT,), jnp.int32),
            pltpu.VMEM((CHT,), jnp.int32),
            pltpu.VMEM((CHT + 16,), jnp.float32),
            pltpu.VMEM((CHT + 16,), jnp.float32),
            pltpu.SemaphoreType.DMA,
        ],
    )
    def k4(yg_hbm, pos_hbm, rw_hbm, out_hbm, bufa, bufb, idx0, idx1, w0v, w1v, sem):
        wid = lax.axis_index("s") * 2 + lax.axis_index("c")
        for half in range(TPW // CHT):
            b = wid * TPW + half * CHT
            pltpu.sync_copy(pos_hbm.at[pl.ds(b, CHT)], idx0)
            pltpu.sync_copy(pos_hbm.at[pl.ds(T + b, CHT)], idx1)
            pltpu.sync_copy(rw_hbm.at[pl.ds(b, CHT)], w0v.at[pl.ds(0, CHT)])
            pltpu.sync_copy(rw_hbm.at[pl.ds(T + b, CHT)], w1v.at[pl.ds(0, CHT)])
            pltpu.async_copy(yg_hbm.at[idx0], bufa, sem).wait()
            pltpu.async_copy(yg_hbm.at[idx1], bufb, sem).wait()

            def token_body(j, _):
                w0 = jnp.full((16,), w0v[pl.ds(j, 16)][0], jnp.float32)
                w1 = jnp.full((16,), w1v[pl.ds(j, 16)][0], jnp.float32)

                def col_body(c, _):
                    for k in range(8):
                        sl = pl.ds(c * 128 + k * 16, 16)
                        bufa[j, sl] = w0 * bufa[j, sl] + w1 * bufb[j, sl]
                    return 0

                lax.fori_loop(0, H // 128, col_body, 0)
                return 0

            lax.fori_loop(0, CHT, token_body, 0)
            pltpu.sync_copy(bufa, out_hbm.at[pl.ds(b, CHT)])

    return k4(yg, pos_flat, rw_flat)


def kernel(hidden_states, router_indices, routing_weights, gate_up_proj,
           gate_up_proj_bias, down_proj, down_proj_bias, lora_gate_up_A,
           lora_gate_up_B, lora_down_A, lora_down_B):
    batch = hidden_states.shape[0]
    hs = hidden_states.reshape(T, H)
    ri = router_indices.astype(jnp.int32)
    rw = routing_weights.astype(jnp.float32)

    x32 = ri.T.reshape(NCH, 1, CH)
    pos, blk = _routing_tc(x32)
    pos_flat = pos.reshape(A)
    blk_flat = blk.reshape(NBA)
    rw_flat = rw.T.reshape(A)

    b1b = gate_up_proj_bias.reshape(E, 1, D)
    bd = down_proj_bias.reshape(E, 1, H)
    dd = jnp.arange(D, dtype=jnp.int32)[:, None]
    jj = jnp.arange(F, dtype=jnp.int32)[None, :]
    sel = (dd == 2 * jj).astype(jnp.bfloat16)

    xg = _dispatch_sc(hs, pos_flat)
    yg = _mlp_tc(blk_flat, xg, gate_up_proj, down_proj, lora_gate_up_A,
                 lora_gate_up_B, lora_down_A, lora_down_B, b1b, bd, sel)
    out = _combine_sc(yg, pos_flat, rw_flat)
    return out.reshape(batch, -1, H)


# speedup vs baseline: 1.1131x; 1.1131x over previous
import functools

import jax
import jax.numpy as jnp
from jax import lax
from jax.experimental import pallas as pl
from jax.experimental.pallas import tpu as pltpu
from jax.experimental.pallas import tpu_sc as plsc

E = 16
H = 1024
F = 1024
D = 2048
R = 4
TOP_K = 2
T = 2048
A = T * TOP_K
TM = 256
NB = E + A // TM
NS = NB * TM
CH = 256
NCH = A // CH
SCALING = 1.0 / R
LIMIT = 7.0
GLU_ALPHA = 1.702

NBA = 40
NW = 32
TPW = T // NW


def _k1_body(x_ref, pos_ref, blk_ref, cnt_ref, carry_ref, st_ref):
    p = pl.program_id(0)
    r = pl.program_id(1)
    x = x_ref[0]
    ei = lax.broadcasted_iota(jnp.int32, (E, CH), 0)
    a16 = (ei == x).astype(jnp.float32)

    @pl.when(p == 0)
    def _():
        ccnt = jnp.sum(a16, axis=1, keepdims=True)
        cnt_ref[...] = jnp.where(r == 0, ccnt, cnt_ref[...] + ccnt)

    @pl.when((p == 1) & (r == 0))
    def _():
        cnt = cnt_ref[...]
        nb = jnp.floor((cnt + (TM - 1.0)) / TM)
        li = lax.broadcasted_iota(jnp.int32, (E, E), 0)
        lj = lax.broadcasted_iota(jnp.int32, (E, E), 1)
        ltri = (lj < li).astype(jnp.float32)
        sb = jnp.dot(ltri, nb, preferred_element_type=jnp.float32)
        st_ref[...] = TM * sb
        carry_ref[...] = jnp.zeros_like(carry_ref)
        bi = lax.broadcasted_iota(jnp.int32, (E, NBA), 1).astype(jnp.float32)
        owns = (sb <= bi).astype(jnp.float32)
        bexp = jnp.sum(owns, axis=0, keepdims=True) - 1.0
        used = jnp.sum(nb)
        ci = lax.broadcasted_iota(jnp.int32, (1, NBA), 1)
        blk_ref[...] = jnp.where(ci == NB, used, bexp).astype(jnp.int32)

    @pl.when(p == 1)
    def _():
        ji = lax.broadcasted_iota(jnp.int32, (CH, CH), 0)
        jc = lax.broadcasted_iota(jnp.int32, (CH, CH), 1)
        utri = (ji < jc).astype(jnp.float32)
        ranks = jnp.dot(a16, utri, preferred_element_type=jnp.float32)
        base = carry_ref[...] + st_ref[...]
        pos = jnp.sum(a16 * (ranks + base), axis=0, keepdims=True)
        pos_ref[...] = pos.astype(jnp.int32).reshape(1, 1, CH)
        carry_ref[...] = carry_ref[...] + jnp.sum(a16, axis=1, keepdims=True)


def _routing_tc(x32, interpret=False):
    return pl.pallas_call(
        _k1_body,
        grid=(2, NCH),
        in_specs=[pl.BlockSpec((1, 1, CH), lambda p, r: (r, 0, 0))],
        out_specs=[
            pl.BlockSpec((1, 1, CH), lambda p, r: (r, 0, 0)),
            pl.BlockSpec((1, NBA), lambda p, r: (0, 0)),
        ],
        out_shape=[
            jax.ShapeDtypeStruct((NCH, 1, CH), jnp.int32),
            jax.ShapeDtypeStruct((1, NBA), jnp.int32),
        ],
        scratch_shapes=[
            pltpu.VMEM((E, 1), jnp.float32),
            pltpu.VMEM((E, 1), jnp.float32),
            pltpu.VMEM((E, 1), jnp.float32),
        ],
        interpret=interpret,
    )(x32)


def _k3_body(blk_ref, x_ref, w1_ref, w2_ref, a1_ref, b1_ref,
             a2_ref, b2_ref, b1b_ref, bd_ref, s_ref, y_ref):
    b = pl.program_id(0)

    @pl.when(b < blk_ref[NB])
    def _():
        x = x_ref[...]
        xb = x.astype(jnp.bfloat16)
        w1 = w1_ref[0].astype(jnp.bfloat16)
        mid = jnp.dot(xb, a1_ref[0].astype(jnp.bfloat16),
                      preferred_element_type=jnp.float32)
        midb = (mid * SCALING).astype(jnp.bfloat16)
        gu = (jnp.dot(xb, w1, preferred_element_type=jnp.float32)
              + jnp.dot(midb, b1_ref[0].astype(jnp.bfloat16),
                        preferred_element_type=jnp.float32)
              + b1b_ref[0])
        ur = jnp.concatenate([gu[:, 1:], gu[:, :1]], axis=1)
        g = jnp.minimum(gu, LIMIT)
        u = jnp.clip(ur, -LIMIT, LIMIT)
        glu = g * (1.0 / (1.0 + jnp.exp(-GLU_ALPHA * g)))
        gatedi = ((u + 1.0) * glu).astype(jnp.bfloat16)
        gatedb = jnp.dot(gatedi, s_ref[...],
                         preferred_element_type=jnp.float32).astype(jnp.bfloat16)
        mid2 = jnp.dot(gatedb, a2_ref[0].astype(jnp.bfloat16),
                       preferred_element_type=jnp.float32)
        mid2b = (mid2 * SCALING).astype(jnp.bfloat16)
        y = (jnp.dot(gatedb, w2_ref[0].astype(jnp.bfloat16),
                     preferred_element_type=jnp.float32)
             + jnp.dot(mid2b, b2_ref[0].astype(jnp.bfloat16),
                       preferred_element_type=jnp.float32)
             + bd_ref[0])
        y_ref[...] = y


def _mlp_tc(blk, xg, w1, w2, a1, b1, a2, b2, b1b, bd, sel, interpret=False):
    eix = lambda b, blk: (blk[b], 0, 0)
    grid_spec = pltpu.PrefetchScalarGridSpec(
        num_scalar_prefetch=1,
        grid=(NB,),
        in_specs=[
            pl.BlockSpec((TM, H), lambda b, blk: (b, 0)),
            pl.BlockSpec((1, H, D), eix),
            pl.BlockSpec((1, F, H), eix),
            pl.BlockSpec((1, H, R), eix),
            pl.BlockSpec((1, R, D), eix),
            pl.BlockSpec((1, F, R), eix),
            pl.BlockSpec((1, R, H), eix),
            pl.BlockSpec((1, 1, D), eix),
            pl.BlockSpec((1, 1, H), eix),
            pl.BlockSpec((D, F), lambda b, blk: (0, 0)),
        ],
        out_specs=pl.BlockSpec((TM, H), lambda b, blk: (b, 0)),
    )
    return pl.pallas_call(
        _k3_body,
        grid_spec=grid_spec,
        out_shape=jax.ShapeDtypeStruct((NS, H), jnp.float32),
        interpret=interpret,
    )(blk, xg, w1, w2, a1, b1, a2, b2, b1b, bd, sel)


def _dispatch_sc(hs, pos_flat):
    mesh = plsc.VectorSubcoreMesh(core_axis_name="c", subcore_axis_name="s")

    @functools.partial(
        pl.kernel,
        mesh=mesh,
        out_type=jax.ShapeDtypeStruct((NS, H), jnp.float32),
        scratch_types=[
            pltpu.VMEM((TPW, H), jnp.float32),
            pltpu.VMEM((TPW,), jnp.int32),
            pltpu.VMEM((TPW,), jnp.int32),
            pltpu.SemaphoreType.DMA,
            pltpu.SemaphoreType.DMA,
        ],
    )
    def k2(hs_hbm, pos_hbm, xg_hbm, rows_v, idx0_v, idx1_v, sem0, sem1):
        wid = lax.axis_index("s") * 2 + lax.axis_index("c")
        base = wid * TPW
        pltpu.sync_copy(pos_hbm.at[pl.ds(base, TPW)], idx0_v)
        pltpu.sync_copy(pos_hbm.at[pl.ds(T + base, TPW)], idx1_v)
        pltpu.sync_copy(hs_hbm.at[pl.ds(base, TPW)], rows_v)
        c0 = pltpu.async_copy(rows_v, xg_hbm.at[idx0_v], sem0)
        c1 = pltpu.async_copy(rows_v, xg_hbm.at[idx1_v], sem1)
        c0.wait()
        c1.wait()

    return k2(hs, pos_flat)


CHT = 32


def _combine_sc(yg, pos_flat, rw_flat):
    mesh = plsc.VectorSubcoreMesh(core_axis_name="c", subcore_axis_name="s")

    @functools.partial(
        pl.kernel,
        mesh=mesh,
        out_type=jax.ShapeDtypeStruct((T, H), jnp.float32),
        scratch_types=[
            pltpu.VMEM((CHT, H), jnp.float32),
            pltpu.VMEM((CHT, H), jnp.float32),
            pltpu.VMEM((CHT,), jnp.int32),
            pltpu.VMEM((CHT,), jnp.int32),
            pltpu.VMEM((CHT + 16,), jnp.float32),
            pltpu.VMEM((CHT + 16,), jnp.float32),
            pltpu.SemaphoreType.DMA,
            pltpu.SemaphoreType.DMA,
        ],
    )
    def k4(yg_hbm, pos_hbm, rw_hbm, out_hbm, bufa, bufb, idx0, idx1, w0v, w1v,
           sem0, sem1):
        wid = lax.axis_index("s") * 2 + lax.axis_index("c")
        for half in range(TPW // CHT):
            b = wid * TPW + half * CHT
            pltpu.sync_copy(pos_hbm.at[pl.ds(b, CHT)], idx0)
            pltpu.sync_copy(pos_hbm.at[pl.ds(T + b, CHT)], idx1)
            c0 = pltpu.async_copy(yg_hbm.at[idx0], bufa, sem0)
            c1 = pltpu.async_copy(yg_hbm.at[idx1], bufb, sem1)
            pltpu.sync_copy(rw_hbm.at[pl.ds(b, CHT)], w0v.at[pl.ds(0, CHT)])
            pltpu.sync_copy(rw_hbm.at[pl.ds(T + b, CHT)], w1v.at[pl.ds(0, CHT)])
            c0.wait()
            c1.wait()

            def token_body(j, _):
                w0 = jnp.full((16,), w0v[pl.ds(j, 16)][0], jnp.float32)
                w1 = jnp.full((16,), w1v[pl.ds(j, 16)][0], jnp.float32)

                def col_body(c, _):
                    for k in range(8):
                        sl = pl.ds(c * 128 + k * 16, 16)
                        bufa[j, sl] = w0 * bufa[j, sl] + w1 * bufb[j, sl]
                    return 0

                lax.fori_loop(0, H // 128, col_body, 0)
                return 0

            lax.fori_loop(0, CHT, token_body, 0)
            pltpu.sync_copy(bufa, out_hbm.at[pl.ds(b, CHT)])

    return k4(yg, pos_flat, rw_flat)


def kernel(hidden_states, router_indices, routing_weights, gate_up_proj,
           gate_up_proj_bias, down_proj, down_proj_bias, lora_gate_up_A,
           lora_gate_up_B, lora_down_A, lora_down_B):
    batch = hidden_states.shape[0]
    hs = hidden_states.reshape(T, H)
    ri = router_indices.astype(jnp.int32)
    rw = routing_weights.astype(jnp.float32)

    x32 = ri.T.reshape(NCH, 1, CH)
    pos, blk = _routing_tc(x32)
    pos_flat = pos.reshape(A)
    blk_flat = blk.reshape(NBA)
    rw_flat = rw.T.reshape(A)

    b1b = gate_up_proj_bias.reshape(E, 1, D)
    bd = down_proj_bias.reshape(E, 1, H)
    dd = jnp.arange(D, dtype=jnp.int32)[:, None]
    jj = jnp.arange(F, dtype=jnp.int32)[None, :]
    sel = (dd == 2 * jj).astype(jnp.bfloat16)

    xg = _dispatch_sc(hs, pos_flat)
    yg = _mlp_tc(blk_flat, xg, gate_up_proj, down_proj, lora_gate_up_A,
                 lora_gate_up_B, lora_down_A, lora_down_B, b1b, bd, sel)
    out = _combine_sc(yg, pos_flat, rw_flat)
    return out.reshape(batch, -1, H)


# speedup vs baseline: 1.1337x; 1.0186x over previous
import functools

import jax
import jax.numpy as jnp
from jax import lax
from jax.experimental import pallas as pl
from jax.experimental.pallas import tpu as pltpu
from jax.experimental.pallas import tpu_sc as plsc

E = 16
H = 1024
F = 1024
D = 2048
R = 4
TOP_K = 2
T = 2048
A = T * TOP_K
TM = 256
NB = E + A // TM
NS = NB * TM
CH = 256
NCH = A // CH
SCALING = 1.0 / R
LIMIT = 7.0
GLU_ALPHA = 1.702

NBA = 40
NW = 32
TPW = T // NW


def _k1_body(x_ref, pos_ref, blk_ref, cnt_ref, carry_ref, st_ref):
    p = pl.program_id(0)
    r = pl.program_id(1)
    x = x_ref[0]
    ei = lax.broadcasted_iota(jnp.int32, (E, CH), 0)
    a16 = (ei == x).astype(jnp.float32)

    @pl.when(p == 0)
    def _():
        ccnt = jnp.sum(a16, axis=1, keepdims=True)
        cnt_ref[...] = jnp.where(r == 0, ccnt, cnt_ref[...] + ccnt)

    @pl.when((p == 1) & (r == 0))
    def _():
        cnt = cnt_ref[...]
        nb = jnp.floor((cnt + (TM - 1.0)) / TM)
        li = lax.broadcasted_iota(jnp.int32, (E, E), 0)
        lj = lax.broadcasted_iota(jnp.int32, (E, E), 1)
        ltri = (lj < li).astype(jnp.float32)
        sb = jnp.dot(ltri, nb, preferred_element_type=jnp.float32)
        st_ref[...] = TM * sb
        carry_ref[...] = jnp.zeros_like(carry_ref)
        bi = lax.broadcasted_iota(jnp.int32, (E, NBA), 1).astype(jnp.float32)
        owns = (sb <= bi).astype(jnp.float32)
        bexp = jnp.sum(owns, axis=0, keepdims=True) - 1.0
        used = jnp.sum(nb)
        ci = lax.broadcasted_iota(jnp.int32, (1, NBA), 1)
        blk_ref[...] = jnp.where(ci == NB, used, bexp).astype(jnp.int32)

    @pl.when(p == 1)
    def _():
        ji = lax.broadcasted_iota(jnp.int32, (CH, CH), 0)
        jc = lax.broadcasted_iota(jnp.int32, (CH, CH), 1)
        utri = (ji < jc).astype(jnp.float32)
        ranks = jnp.dot(a16, utri, preferred_element_type=jnp.float32)
        base = carry_ref[...] + st_ref[...]
        pos = jnp.sum(a16 * (ranks + base), axis=0, keepdims=True)
        pos_ref[...] = pos.astype(jnp.int32).reshape(1, 1, CH)
        carry_ref[...] = carry_ref[...] + jnp.sum(a16, axis=1, keepdims=True)


def _routing_tc(x32, interpret=False):
    return pl.pallas_call(
        _k1_body,
        grid=(2, NCH),
        in_specs=[pl.BlockSpec((1, 1, CH), lambda p, r: (r, 0, 0))],
        out_specs=[
            pl.BlockSpec((1, 1, CH), lambda p, r: (r, 0, 0)),
            pl.BlockSpec((1, NBA), lambda p, r: (0, 0)),
        ],
        out_shape=[
            jax.ShapeDtypeStruct((NCH, 1, CH), jnp.int32),
            jax.ShapeDtypeStruct((1, NBA), jnp.int32),
        ],
        scratch_shapes=[
            pltpu.VMEM((E, 1), jnp.float32),
            pltpu.VMEM((E, 1), jnp.float32),
            pltpu.VMEM((E, 1), jnp.float32),
        ],
        interpret=interpret,
    )(x32)


def _k3_body(blk_ref, x_ref, w1_ref, w2_ref, a1_ref, b1_ref,
             a2_ref, b2_ref, b1b_ref, bd_ref, s_ref, y_ref):
    b = pl.program_id(0)

    @pl.when(b < blk_ref[NB])
    def _():
        x = x_ref[...]
        xb = x.astype(jnp.bfloat16)
        w1 = w1_ref[0].astype(jnp.bfloat16)
        mid = jnp.dot(xb, a1_ref[0].astype(jnp.bfloat16),
                      preferred_element_type=jnp.float32)
        midb = (mid * SCALING).astype(jnp.bfloat16)
        gu = (jnp.dot(xb, w1, preferred_element_type=jnp.float32)
              + jnp.dot(midb, b1_ref[0].astype(jnp.bfloat16),
                        preferred_element_type=jnp.float32)
              + b1b_ref[0])
        ur = jnp.concatenate([gu[:, 1:], gu[:, :1]], axis=1)
        g = jnp.minimum(gu, LIMIT)
        u = jnp.clip(ur, -LIMIT, LIMIT)
        glu = g * (1.0 / (1.0 + jnp.exp(-GLU_ALPHA * g)))
        gatedi = ((u + 1.0) * glu).astype(jnp.bfloat16)
        gatedb = jnp.dot(gatedi, s_ref[...],
                         preferred_element_type=jnp.float32).astype(jnp.bfloat16)
        mid2 = jnp.dot(gatedb, a2_ref[0].astype(jnp.bfloat16),
                       preferred_element_type=jnp.float32)
        mid2b = (mid2 * SCALING).astype(jnp.bfloat16)
        y = (jnp.dot(gatedb, w2_ref[0].astype(jnp.bfloat16),
                     preferred_element_type=jnp.float32)
             + jnp.dot(mid2b, b2_ref[0].astype(jnp.bfloat16),
                       preferred_element_type=jnp.float32)
             + bd_ref[0])
        y_ref[...] = y


def _mlp_tc(blk, xg, w1, w2, a1, b1, a2, b2, b1b, bd, sel, interpret=False):
    eix = lambda b, blk: (blk[b], 0, 0)
    grid_spec = pltpu.PrefetchScalarGridSpec(
        num_scalar_prefetch=1,
        grid=(NB,),
        in_specs=[
            pl.BlockSpec((TM, H), lambda b, blk: (b, 0)),
            pl.BlockSpec((1, H, D), eix),
            pl.BlockSpec((1, F, H), eix),
            pl.BlockSpec((1, H, R), eix),
            pl.BlockSpec((1, R, D), eix),
            pl.BlockSpec((1, F, R), eix),
            pl.BlockSpec((1, R, H), eix),
            pl.BlockSpec((1, 1, D), eix),
            pl.BlockSpec((1, 1, H), eix),
            pl.BlockSpec((D, F), lambda b, blk: (0, 0)),
        ],
        out_specs=pl.BlockSpec((TM, H), lambda b, blk: (b, 0)),
    )
    return pl.pallas_call(
        _k3_body,
        grid_spec=grid_spec,
        out_shape=jax.ShapeDtypeStruct((NS, H), jnp.float32),
        interpret=interpret,
    )(blk, xg, w1, w2, a1, b1, a2, b2, b1b, bd, sel)


def _dispatch_sc(hs, pos_flat):
    mesh = plsc.VectorSubcoreMesh(core_axis_name="c", subcore_axis_name="s")

    @functools.partial(
        pl.kernel,
        mesh=mesh,
        out_type=jax.ShapeDtypeStruct((NS, H), jnp.float32),
        scratch_types=[
            pltpu.VMEM((TPW, H), jnp.float32),
            pltpu.VMEM((TPW,), jnp.int32),
            pltpu.VMEM((TPW,), jnp.int32),
            pltpu.SemaphoreType.DMA,
            pltpu.SemaphoreType.DMA,
        ],
    )
    def k2(hs_hbm, pos_hbm, xg_hbm, rows_v, idx0_v, idx1_v, sem0, sem1):
        wid = lax.axis_index("s") * 2 + lax.axis_index("c")
        base = wid * TPW
        pltpu.sync_copy(pos_hbm.at[pl.ds(base, TPW)], idx0_v)
        pltpu.sync_copy(pos_hbm.at[pl.ds(T + base, TPW)], idx1_v)
        pltpu.sync_copy(hs_hbm.at[pl.ds(base, TPW)], rows_v)
        c0 = pltpu.async_copy(rows_v, xg_hbm.at[idx0_v], sem0)
        c1 = pltpu.async_copy(rows_v, xg_hbm.at[idx1_v], sem1)
        c0.wait()
        c1.wait()

    return k2(hs, pos_flat)


CHT = 16
NCHK = TPW // CHT


def _combine_sc(yg, pos_flat, rw_flat):
    mesh = plsc.VectorSubcoreMesh(core_axis_name="c", subcore_axis_name="s")

    @functools.partial(
        pl.kernel,
        mesh=mesh,
        out_type=jax.ShapeDtypeStruct((T, H), jnp.float32),
        scratch_types=[
            pltpu.VMEM((CHT, H), jnp.float32),
            pltpu.VMEM((CHT, H), jnp.float32),
            pltpu.VMEM((CHT, H), jnp.float32),
            pltpu.VMEM((CHT, H), jnp.float32),
            pltpu.VMEM((TPW,), jnp.int32),
            pltpu.VMEM((TPW,), jnp.int32),
            pltpu.VMEM((TPW + 16,), jnp.float32),
            pltpu.VMEM((TPW + 16,), jnp.float32),
            pltpu.SemaphoreType.DMA,
            pltpu.SemaphoreType.DMA,
            pltpu.SemaphoreType.DMA,
            pltpu.SemaphoreType.DMA,
            pltpu.SemaphoreType.DMA,
            pltpu.SemaphoreType.DMA,
        ],
    )
    def k4(yg_hbm, pos_hbm, rw_hbm, out_hbm, bufa0, bufb0, bufa1, bufb1,
           idxa, idxb, wav, wbv, sa0, sb0, sa1, sb1, st0, st1):
        wid = lax.axis_index("s") * 2 + lax.axis_index("c")
        base = wid * TPW
        bufa = [bufa0, bufa1]
        bufb = [bufb0, bufb1]
        sa = [sa0, sa1]
        sb = [sb0, sb1]
        st = [st0, st1]
        pltpu.sync_copy(pos_hbm.at[pl.ds(base, TPW)], idxa)
        pltpu.sync_copy(pos_hbm.at[pl.ds(T + base, TPW)], idxb)
        pltpu.sync_copy(rw_hbm.at[pl.ds(base, TPW)], wav.at[pl.ds(0, TPW)])
        pltpu.sync_copy(rw_hbm.at[pl.ds(T + base, TPW)], wbv.at[pl.ds(0, TPW)])

        def fire(c):
            s = c % 2
            ga = pltpu.async_copy(yg_hbm.at[idxa.at[pl.ds(c * CHT, CHT)]],
                                  bufa[s], sa[s])
            gb = pltpu.async_copy(yg_hbm.at[idxb.at[pl.ds(c * CHT, CHT)]],
                                  bufb[s], sb[s])
            return ga, gb

        pend = {0: fire(0)}
        stores = [None, None]
        for c in range(NCHK):
            s = c % 2
            if c + 1 < NCHK:
                if stores[(c + 1) % 2] is not None:
                    stores[(c + 1) % 2].wait()
                    stores[(c + 1) % 2] = None
                pend[c + 1] = fire(c + 1)
            ga, gb = pend.pop(c)
            ga.wait()
            gb.wait()
            ba = bufa[s]
            bb = bufb[s]

            def token_body(j, _, ba=ba, bb=bb, woff=c * CHT):
                w0 = jnp.full((16,), wav[pl.ds(woff + j, 16)][0], jnp.float32)
                w1 = jnp.full((16,), wbv[pl.ds(woff + j, 16)][0], jnp.float32)

                def col_body(cc, _):
                    for k in range(8):
                        sl = pl.ds(cc * 128 + k * 16, 16)
                        ba[j, sl] = w0 * ba[j, sl] + w1 * bb[j, sl]
                    return 0

                lax.fori_loop(0, H // 128, col_body, 0)
                return 0

            lax.fori_loop(0, CHT, token_body, 0)
            stores[s] = pltpu.async_copy(
                ba, out_hbm.at[pl.ds(base + c * CHT, CHT)], st[s])
        for s in range(2):
            if stores[s] is not None:
                stores[s].wait()

    return k4(yg, pos_flat, rw_flat)


def kernel(hidden_states, router_indices, routing_weights, gate_up_proj,
           gate_up_proj_bias, down_proj, down_proj_bias, lora_gate_up_A,
           lora_gate_up_B, lora_down_A, lora_down_B):
    batch = hidden_states.shape[0]
    hs = hidden_states.reshape(T, H)
    ri = router_indices.astype(jnp.int32)
    rw = routing_weights.astype(jnp.float32)

    x32 = ri.T.reshape(NCH, 1, CH)
    pos, blk = _routing_tc(x32)
    pos_flat = pos.reshape(A)
    blk_flat = blk.reshape(NBA)
    rw_flat = rw.T.reshape(A)

    b1b = gate_up_proj_bias.reshape(E, 1, D)
    bd = down_proj_bias.reshape(E, 1, H)
    dd = jnp.arange(D, dtype=jnp.int32)[:, None]
    jj = jnp.arange(F, dtype=jnp.int32)[None, :]
    sel = (dd == 2 * jj).astype(jnp.bfloat16)

    xg = _dispatch_sc(hs, pos_flat)
    yg = _mlp_tc(blk_flat, xg, gate_up_proj, down_proj, lora_gate_up_A,
                 lora_gate_up_B, lora_down_A, lora_down_B, b1b, bd, sel)
    out = _combine_sc(yg, pos_flat, rw_flat)
    return out.reshape(batch, -1, H)


# speedup vs baseline: 1.1533x; 1.0173x over previous
import functools

import numpy as np

import jax
import jax.numpy as jnp
from jax import lax
from jax.experimental import pallas as pl
from jax.experimental.pallas import tpu as pltpu
from jax.experimental.pallas import tpu_sc as plsc

E = 16
H = 1024
F = 1024
D = 2048
R = 4
TOP_K = 2
T = 2048
A = T * TOP_K
TM = 256
NB = E + A // TM
NS = NB * TM
CH = 256
NCH = A // CH
SCALING = 1.0 / R
LIMIT = 7.0
GLU_ALPHA = 1.702

NBA = 40
NW = 32
TPW = T // NW

_SEL = (np.arange(D)[:, None] == 2 * np.arange(F)[None, :]).astype(
    jnp.bfloat16)


def _k1_body(x_ref, pos_ref, blk_ref, cnt_ref, carry_ref, st_ref):
    p = pl.program_id(0)
    r = pl.program_id(1)
    x = x_ref[0]
    ei = lax.broadcasted_iota(jnp.int32, (E, CH), 0)
    a16 = (ei == x).astype(jnp.float32)

    @pl.when(p == 0)
    def _():
        ccnt = jnp.sum(a16, axis=1, keepdims=True)
        cnt_ref[...] = jnp.where(r == 0, ccnt, cnt_ref[...] + ccnt)

    @pl.when((p == 1) & (r == 0))
    def _():
        cnt = cnt_ref[...]
        nb = jnp.floor((cnt + (TM - 1.0)) / TM)
        li = lax.broadcasted_iota(jnp.int32, (E, E), 0)
        lj = lax.broadcasted_iota(jnp.int32, (E, E), 1)
        ltri = (lj < li).astype(jnp.float32)
        sb = jnp.dot(ltri, nb, preferred_element_type=jnp.float32)
        st_ref[...] = TM * sb
        carry_ref[...] = jnp.zeros_like(carry_ref)
        bi = lax.broadcasted_iota(jnp.int32, (E, NBA), 1).astype(jnp.float32)
        owns = (sb <= bi).astype(jnp.float32)
        bexp = jnp.sum(owns, axis=0, keepdims=True) - 1.0
        used = jnp.sum(nb)
        ci = lax.broadcasted_iota(jnp.int32, (1, NBA), 1)
        blk_ref[...] = jnp.where(ci == NB, used, bexp).astype(jnp.int32)

    @pl.when(p == 1)
    def _():
        ji = lax.broadcasted_iota(jnp.int32, (CH, CH), 0)
        jc = lax.broadcasted_iota(jnp.int32, (CH, CH), 1)
        utri = (ji < jc).astype(jnp.float32)
        ranks = jnp.dot(a16, utri, preferred_element_type=jnp.float32)
        base = carry_ref[...] + st_ref[...]
        pos = jnp.sum(a16 * (ranks + base), axis=0, keepdims=True)
        pos_ref[...] = pos.astype(jnp.int32).reshape(1, 1, CH)
        carry_ref[...] = carry_ref[...] + jnp.sum(a16, axis=1, keepdims=True)


def _routing_tc(x32, interpret=False):
    return pl.pallas_call(
        _k1_body,
        grid=(2, NCH),
        in_specs=[pl.BlockSpec((1, 1, CH), lambda p, r: (r, 0, 0))],
        out_specs=[
            pl.BlockSpec((1, 1, CH), lambda p, r: (r, 0, 0)),
            pl.BlockSpec((1, NBA), lambda p, r: (0, 0)),
        ],
        out_shape=[
            jax.ShapeDtypeStruct((NCH, 1, CH), jnp.int32),
            jax.ShapeDtypeStruct((1, NBA), jnp.int32),
        ],
        scratch_shapes=[
            pltpu.VMEM((E, 1), jnp.float32),
            pltpu.VMEM((E, 1), jnp.float32),
            pltpu.VMEM((E, 1), jnp.float32),
        ],
        interpret=interpret,
    )(x32)


def _k3_body(blk_ref, x_ref, w1_ref, w2_ref, a1_ref, b1_ref,
             a2_ref, b2_ref, b1b_ref, bd_ref, s_ref, y_ref):
    b = pl.program_id(0)

    @pl.when(b < blk_ref[NB])
    def _():
        x = x_ref[...]
        xb = x.astype(jnp.bfloat16)
        w1 = w1_ref[0].astype(jnp.bfloat16)
        mid = jnp.dot(xb, a1_ref[0].astype(jnp.bfloat16),
                      preferred_element_type=jnp.float32)
        midb = (mid * SCALING).astype(jnp.bfloat16)
        gu = (jnp.dot(xb, w1, preferred_element_type=jnp.float32)
              + jnp.dot(midb, b1_ref[0].astype(jnp.bfloat16),
                        preferred_element_type=jnp.float32)
              + b1b_ref[0])
        ur = jnp.concatenate([gu[:, 1:], gu[:, :1]], axis=1)
        g = jnp.minimum(gu, LIMIT)
        u = jnp.clip(ur, -LIMIT, LIMIT)
        glu = g * (1.0 / (1.0 + jnp.exp(-GLU_ALPHA * g)))
        gatedi = ((u + 1.0) * glu).astype(jnp.bfloat16)
        gatedb = jnp.dot(gatedi, s_ref[...],
                         preferred_element_type=jnp.float32).astype(jnp.bfloat16)
        mid2 = jnp.dot(gatedb, a2_ref[0].astype(jnp.bfloat16),
                       preferred_element_type=jnp.float32)
        mid2b = (mid2 * SCALING).astype(jnp.bfloat16)
        y = (jnp.dot(gatedb, w2_ref[0].astype(jnp.bfloat16),
                     preferred_element_type=jnp.float32)
             + jnp.dot(mid2b, b2_ref[0].astype(jnp.bfloat16),
                       preferred_element_type=jnp.float32)
             + bd_ref[0])
        y_ref[...] = y


def _mlp_tc(blk, xg, w1, w2, a1, b1, a2, b2, b1b, bd, sel, interpret=False):
    eix = lambda b, blk: (blk[b], 0, 0)
    grid_spec = pltpu.PrefetchScalarGridSpec(
        num_scalar_prefetch=1,
        grid=(NB,),
        in_specs=[
            pl.BlockSpec((TM, H), lambda b, blk: (b, 0)),
            pl.BlockSpec((1, H, D), eix),
            pl.BlockSpec((1, F, H), eix),
            pl.BlockSpec((1, H, R), eix),
            pl.BlockSpec((1, R, D), eix),
            pl.BlockSpec((1, F, R), eix),
            pl.BlockSpec((1, R, H), eix),
            pl.BlockSpec((1, 1, D), eix),
            pl.BlockSpec((1, 1, H), eix),
            pl.BlockSpec((D, F), lambda b, blk: (0, 0)),
        ],
        out_specs=pl.BlockSpec((TM, H), lambda b, blk: (b, 0)),
    )
    return pl.pallas_call(
        _k3_body,
        grid_spec=grid_spec,
        out_shape=jax.ShapeDtypeStruct((NS, H), jnp.float32),
        interpret=interpret,
    )(blk, xg, w1, w2, a1, b1, a2, b2, b1b, bd, sel)


def _dispatch_sc(hs, pos_flat):
    mesh = plsc.VectorSubcoreMesh(core_axis_name="c", subcore_axis_name="s")

    @functools.partial(
        pl.kernel,
        mesh=mesh,
        out_type=jax.ShapeDtypeStruct((NS, H), jnp.float32),
        scratch_types=[
            pltpu.VMEM((TPW, H), jnp.float32),
            pltpu.VMEM((TPW,), jnp.int32),
            pltpu.VMEM((TPW,), jnp.int32),
            pltpu.SemaphoreType.DMA,
            pltpu.SemaphoreType.DMA,
        ],
    )
    def k2(hs_hbm, pos_hbm, xg_hbm, rows_v, idx0_v, idx1_v, sem0, sem1):
        wid = lax.axis_index("s") * 2 + lax.axis_index("c")
        base = wid * TPW
        pltpu.sync_copy(pos_hbm.at[pl.ds(base, TPW)], idx0_v)
        pltpu.sync_copy(pos_hbm.at[pl.ds(T + base, TPW)], idx1_v)
        pltpu.sync_copy(hs_hbm.at[pl.ds(base, TPW)], rows_v)
        c0 = pltpu.async_copy(rows_v, xg_hbm.at[idx0_v], sem0)
        c1 = pltpu.async_copy(rows_v, xg_hbm.at[idx1_v], sem1)
        c0.wait()
        c1.wait()

    return k2(hs, pos_flat)


CHT = 16
NCHK = TPW // CHT


def _combine_sc(yg, pos_flat, rw_flat):
    mesh = plsc.VectorSubcoreMesh(core_axis_name="c", subcore_axis_name="s")

    @functools.partial(
        pl.kernel,
        mesh=mesh,
        out_type=jax.ShapeDtypeStruct((T, H), jnp.float32),
        scratch_types=[
            pltpu.VMEM((CHT, H), jnp.float32),
            pltpu.VMEM((CHT, H), jnp.float32),
            pltpu.VMEM((CHT, H), jnp.float32),
            pltpu.VMEM((CHT, H), jnp.float32),
            pltpu.VMEM((TPW,), jnp.int32),
            pltpu.VMEM((TPW,), jnp.int32),
            pltpu.VMEM((TPW + 16,), jnp.float32),
            pltpu.VMEM((TPW + 16,), jnp.float32),
            pltpu.SemaphoreType.DMA,
            pltpu.SemaphoreType.DMA,
            pltpu.SemaphoreType.DMA,
            pltpu.SemaphoreType.DMA,
            pltpu.SemaphoreType.DMA,
            pltpu.SemaphoreType.DMA,
        ],
    )
    def k4(yg_hbm, pos_hbm, rw_hbm, out_hbm, bufa0, bufb0, bufa1, bufb1,
           idxa, idxb, wav, wbv, sa0, sb0, sa1, sb1, st0, st1):
        wid = lax.axis_index("s") * 2 + lax.axis_index("c")
        base = wid * TPW
        bufa = [bufa0, bufa1]
        bufb = [bufb0, bufb1]
        sa = [sa0, sa1]
        sb = [sb0, sb1]
        st = [st0, st1]
        pltpu.sync_copy(pos_hbm.at[pl.ds(base, TPW)], idxa)
        pltpu.sync_copy(pos_hbm.at[pl.ds(T + base, TPW)], idxb)
        pltpu.sync_copy(rw_hbm.at[pl.ds(base, TPW)], wav.at[pl.ds(0, TPW)])
        pltpu.sync_copy(rw_hbm.at[pl.ds(T + base, TPW)], wbv.at[pl.ds(0, TPW)])

        def fire(c):
            s = c % 2
            ga = pltpu.async_copy(yg_hbm.at[idxa.at[pl.ds(c * CHT, CHT)]],
                                  bufa[s], sa[s])
            gb = pltpu.async_copy(yg_hbm.at[idxb.at[pl.ds(c * CHT, CHT)]],
                                  bufb[s], sb[s])
            return ga, gb

        pend = {0: fire(0)}
        stores = [None, None]
        for c in range(NCHK):
            s = c % 2
            if c + 1 < NCHK:
                if stores[(c + 1) % 2] is not None:
                    stores[(c + 1) % 2].wait()
                    stores[(c + 1) % 2] = None
                pend[c + 1] = fire(c + 1)
            ga, gb = pend.pop(c)
            ga.wait()
            gb.wait()
            ba = bufa[s]
            bb = bufb[s]

            def token_body(j, _, ba=ba, bb=bb, woff=c * CHT):
                w0 = jnp.full((16,), wav[pl.ds(woff + j, 16)][0], jnp.float32)
                w1 = jnp.full((16,), wbv[pl.ds(woff + j, 16)][0], jnp.float32)

                def col_body(cc, _):
                    for k in range(8):
                        sl = pl.ds(cc * 128 + k * 16, 16)
                        ba[j, sl] = w0 * ba[j, sl] + w1 * bb[j, sl]
                    return 0

                lax.fori_loop(0, H // 128, col_body, 0)
                return 0

            lax.fori_loop(0, CHT, token_body, 0)
            stores[s] = pltpu.async_copy(
                ba, out_hbm.at[pl.ds(base + c * CHT, CHT)], st[s])
        for s in range(2):
            if stores[s] is not None:
                stores[s].wait()

    return k4(yg, pos_flat, rw_flat)


def kernel(hidden_states, router_indices, routing_weights, gate_up_proj,
           gate_up_proj_bias, down_proj, down_proj_bias, lora_gate_up_A,
           lora_gate_up_B, lora_down_A, lora_down_B):
    batch = hidden_states.shape[0]
    hs = hidden_states.reshape(T, H)
    ri = router_indices.astype(jnp.int32)
    rw = routing_weights.astype(jnp.float32)

    x32 = ri.T.reshape(NCH, 1, CH)
    pos, blk = _routing_tc(x32)
    pos_flat = pos.reshape(A)
    blk_flat = blk.reshape(NBA)
    rw_flat = rw.T.reshape(A)

    b1b = gate_up_proj_bias.reshape(E, 1, D)
    bd = down_proj_bias.reshape(E, 1, H)
    sel = jnp.asarray(_SEL)

    xg = _dispatch_sc(hs, pos_flat)
    yg = _mlp_tc(blk_flat, xg, gate_up_proj, down_proj, lora_gate_up_A,
                 lora_gate_up_B, lora_down_A, lora_down_B, b1b, bd, sel)
    out = _combine_sc(yg, pos_flat, rw_flat)
    return out.reshape(batch, -1, H)
